# Initial kernel scaffold; baseline (speedup 1.0000x reference)
import jax
import jax.numpy as jnp
from jax import lax
from jax.experimental import pallas as pl
from jax.experimental.pallas import tpu as pltpu
from jax.experimental.pallas import tpu_sc as plsc

VOCAB = 1000000
EMBED_DIM = 32
BATCH = 16384
HIST = 200

NC = 2
NS = 16
NW = NC * NS

N = BATCH * HIST
PER_W = N // NW
CH = 2048
K = CH // 128
STEPS = PER_W // CH


def _body(x_hbm, table_hbm, out_hbm, idx_v, rows_v, sem):
    wid = lax.axis_index("s") * NC + lax.axis_index("c")
    base = wid * PER_W

    def step_fn(step, carry):
        pltpu.sync_copy(x_hbm.at[wid, step], idx_v)
        copies = []
        for j in range(K):
            copies.append(
                pltpu.async_copy(
                    table_hbm.at[idx_v.at[j]],
                    rows_v.at[pl.ds(j * 128, 128)],
                    sem,
                )
            )
        for c in copies:
            c.wait()
        pltpu.sync_copy(rows_v, out_hbm.at[pl.ds(base + step * CH, CH)])
        return carry

    lax.fori_loop(0, STEPS, step_fn, 0)


@jax.jit
def _lookup(x_r, table):
    mesh = plsc.VectorSubcoreMesh(core_axis_name="c", subcore_axis_name="s")
    f = pl.kernel(
        _body,
        out_type=jax.ShapeDtypeStruct((N, EMBED_DIM), jnp.float32),
        mesh=mesh,
        scratch_types=[
            pltpu.VMEM((K, 128), jnp.int32),
            pltpu.VMEM((CH, EMBED_DIM), jnp.float32),
            pltpu.SemaphoreType.DMA,
        ],
    )
    return f(x_r, table)


def kernel(x, table):
    x_r = x.reshape(NW, STEPS, K, 128).astype(jnp.int32)
    out = _lookup(x_r, table)
    return out.reshape(BATCH, HIST, EMBED_DIM)


# speedup vs baseline: 4.9442x; 4.9442x over previous
import jax
import jax.numpy as jnp
from jax import lax
from jax.experimental import pallas as pl
from jax.experimental.pallas import tpu as pltpu
from jax.experimental.pallas import tpu_sc as plsc

VOCAB = 1000000
EMBED_DIM = 32
BATCH = 16384
HIST = 200

NC = 2
NS = 16
NW = NC * NS

N = BATCH * HIST
PER_W = N // NW
CH = 2048
K = CH // 128
STEPS = PER_W // CH


def _body(x_hbm, table_hbm, out_hbm, idx_v, rows_v, sem):
    wid = lax.axis_index("s") * NC + lax.axis_index("c")
    base = wid * PER_W

    def step_fn(step, carry):
        pltpu.sync_copy(x_hbm.at[wid, step], idx_v)
        copies = []
        for j in range(K):
            copies.append(
                pltpu.async_copy(
                    table_hbm.at[idx_v.at[j]],
                    rows_v.at[pl.ds(j * 128, 128)],
                    sem,
                )
            )
        for c in copies:
            c.wait()
        pltpu.sync_copy(rows_v, out_hbm.at[pl.ds(base + step * CH, CH)])
        return carry

    lax.fori_loop(0, STEPS, step_fn, 0)


@jax.jit
def _lookup(x_r, table):
    mesh = plsc.VectorSubcoreMesh(core_axis_name="c", subcore_axis_name="s")
    f = pl.kernel(
        _body,
        out_type=jax.ShapeDtypeStruct((N, EMBED_DIM), jnp.float32),
        mesh=mesh,
        scratch_types=[
            pltpu.VMEM((K, 128), jnp.int32),
            pltpu.VMEM((CH, EMBED_DIM), jnp.float32),
            pltpu.SemaphoreType.DMA,
        ],
        compiler_params=pltpu.CompilerParams(use_tc_tiling_on_sc=False),
    )
    return f(x_r, table)


def kernel(x, table):
    x_r = x.reshape(NW, STEPS, K, 128).astype(jnp.int32)
    out = _lookup(x_r, table)
    return out.reshape(BATCH, HIST, EMBED_DIM)


# speedup vs baseline: 4.9675x; 1.0047x over previous
import jax
import jax.numpy as jnp
from jax import lax
from jax.experimental import pallas as pl
from jax.experimental.pallas import tpu as pltpu
from jax.experimental.pallas import tpu_sc as plsc

VOCAB = 1000000
EMBED_DIM = 32
BATCH = 16384
HIST = 200

NC = 2
NS = 16
NW = NC * NS

N = BATCH * HIST
PER_W = N // NW
CH = 1280
K = CH // 128
STEPS = PER_W // CH


def _fire(x_hbm, table_hbm, idx_v, rows_v, sems, wid, s, slot):
    pltpu.sync_copy(x_hbm.at[wid, s], idx_v.at[slot])
    for j in range(K):
        pltpu.async_copy(
            table_hbm.at[idx_v.at[slot, j]],
            rows_v.at[slot, pl.ds(j * 128, 128)],
            sems.at[slot],
        )


def _drain(table_hbm, idx_v, rows_v, sems, slot):
    for j in range(K):
        pltpu.make_async_copy(
            table_hbm.at[idx_v.at[slot, j]],
            rows_v.at[slot, pl.ds(j * 128, 128)],
            sems.at[slot],
        ).wait()


def _body(x_hbm, table_hbm, out_hbm, idx_v, rows_v, sems):
    wid = lax.axis_index("s") * NC + lax.axis_index("c")
    base = wid * PER_W

    _fire(x_hbm, table_hbm, idx_v, rows_v, sems, wid, 0, 0)

    def pair_fn(g, carry):
        for b in range(2):
            s = 2 * g + b

            @pl.when(s + 1 < STEPS)
            def _():
                _fire(x_hbm, table_hbm, idx_v, rows_v, sems, wid, s + 1, 1 - b)

            _drain(table_hbm, idx_v, rows_v, sems, b)
            pltpu.sync_copy(rows_v.at[b], out_hbm.at[pl.ds(base + s * CH, CH)])
        return carry

    lax.fori_loop(0, STEPS // 2, pair_fn, 0)


@jax.jit
def _lookup(x_r, table):
    mesh = plsc.VectorSubcoreMesh(core_axis_name="c", subcore_axis_name="s")
    f = pl.kernel(
        _body,
        out_type=jax.ShapeDtypeStruct((N, EMBED_DIM), jnp.float32),
        mesh=mesh,
        scratch_types=[
            pltpu.VMEM((2, K, 128), jnp.int32),
            pltpu.VMEM((2, CH, EMBED_DIM), jnp.float32),
            pltpu.SemaphoreType.DMA((2,)),
        ],
        compiler_params=pltpu.CompilerParams(use_tc_tiling_on_sc=False),
    )
    return f(x_r, table)


def kernel(x, table):
    x_r = x.reshape(NW, STEPS, K, 128).astype(jnp.int32)
    out = _lookup(x_r, table)
    return out.reshape(BATCH, HIST, EMBED_DIM)


# speedup vs baseline: 4.9805x; 1.0026x over previous
import jax
import jax.numpy as jnp
from jax import lax
from jax.experimental import pallas as pl
from jax.experimental.pallas import tpu as pltpu
from jax.experimental.pallas import tpu_sc as plsc

VOCAB = 1000000
EMBED_DIM = 32
BATCH = 16384
HIST = 200

NC = 2
NS = 16
NW = NC * NS

N = BATCH * HIST
PER_W = N // NW
CH = 1600
STEPS = PER_W // CH


def _fire(x_hbm, table_hbm, idx_v, rows_v, sems, wid, s, slot):
    pltpu.sync_copy(x_hbm.at[wid, s], idx_v.at[slot])
    pltpu.async_copy(
        table_hbm.at[idx_v.at[slot]],
        rows_v.at[slot],
        sems.at[slot],
    )


def _drain(table_hbm, idx_v, rows_v, sems, slot):
    pltpu.make_async_copy(
        table_hbm.at[idx_v.at[slot]],
        rows_v.at[slot],
        sems.at[slot],
    ).wait()


def _body(x_hbm, table_hbm, out_hbm, idx_v, rows_v, sems):
    wid = lax.axis_index("s") * NC + lax.axis_index("c")
    base = wid * PER_W

    _fire(x_hbm, table_hbm, idx_v, rows_v, sems, wid, 0, 0)

    def pair_fn(g, carry):
        for b in range(2):
            s = 2 * g + b

            @pl.when(s + 1 < STEPS)
            def _():
                _fire(x_hbm, table_hbm, idx_v, rows_v, sems, wid, s + 1, 1 - b)

            _drain(table_hbm, idx_v, rows_v, sems, b)
            pltpu.sync_copy(rows_v.at[b], out_hbm.at[pl.ds(base + s * CH, CH)])
        return carry

    lax.fori_loop(0, STEPS // 2, pair_fn, 0)


@jax.jit
def _lookup(x_r, table):
    mesh = plsc.VectorSubcoreMesh(core_axis_name="c", subcore_axis_name="s")
    f = pl.kernel(
        _body,
        out_type=jax.ShapeDtypeStruct((N, EMBED_DIM), jnp.float32),
        mesh=mesh,
        scratch_types=[
            pltpu.VMEM((2, CH), jnp.int32),
            pltpu.VMEM((2, CH, EMBED_DIM), jnp.float32),
            pltpu.SemaphoreType.DMA((2,)),
        ],
        compiler_params=pltpu.CompilerParams(use_tc_tiling_on_sc=False),
    )
    return f(x_r, table)


def kernel(x, table):
    x_r = x.reshape(NW, STEPS, CH).astype(jnp.int32)
    out = _lookup(x_r, table)
    return out.reshape(BATCH, HIST, EMBED_DIM)
